# baseline (speedup 1.0000x reference)
import jax
import jax.numpy as jnp
from jax.experimental import pallas as pl

_TOKENS = 16384
_PATHS = 64
_BLOCK_ROWS = 2048


def _body(s_ref, o_ref):
    s = s_ref[...]
    m = (s > 0.0).astype(jnp.int32)
    rowsum = jnp.sum(m, axis=1, keepdims=True)
    resid = (rowsum == 0).astype(jnp.int32)
    col0 = jax.lax.broadcasted_iota(jnp.int32, m.shape, 1) == 0
    o_ref[...] = m + jnp.where(col0, resid, 0)


def kernel(score):
    return pl.pallas_call(
        _body,
        out_shape=jax.ShapeDtypeStruct((_TOKENS, _PATHS), jnp.int32),
        grid=(_TOKENS // _BLOCK_ROWS,),
        in_specs=[pl.BlockSpec((_BLOCK_ROWS, _PATHS), lambda i: (i, 0))],
        out_specs=pl.BlockSpec((_BLOCK_ROWS, _PATHS), lambda i: (i, 0)),
    )(score)
